# Initial kernel scaffold; baseline (speedup 1.0000x reference)
import jax
import jax.numpy as jnp
from jax.experimental import pallas as pl


def kernel(input, adj, weight, bias):
    raise NotImplementedError("write your pallas kernel here")


# speedup vs baseline: 1.1033x; 1.1033x over previous
import functools

import jax
import jax.numpy as jnp
from jax.experimental import pallas as pl
from jax.experimental.pallas import tpu as pltpu


def _gcn_body(x_ref, w_ref, b_ref, adj_ref, out_ref, supp_ref):
    @pl.when(pl.program_id(1) == 0)
    def _():
        supp_ref[...] = jnp.dot(
            x_ref[0], w_ref[...], preferred_element_type=jnp.float32
        )

    out_ref[0] = (
        jnp.dot(adj_ref[0], supp_ref[...], preferred_element_type=jnp.float32)
        + b_ref[...]
    )


@functools.partial(jax.jit, static_argnames=())
def kernel(input, adj, weight, bias):
    B, N, IN = input.shape
    OUT = weight.shape[1]
    BLK_I = min(256, N)

    grid = (B, N // BLK_I)
    out = pl.pallas_call(
        _gcn_body,
        grid=grid,
        in_specs=[
            pl.BlockSpec((1, N, IN), lambda b, i: (b, 0, 0)),
            pl.BlockSpec((IN, OUT), lambda b, i: (0, 0)),
            pl.BlockSpec((1, OUT), lambda b, i: (0, 0)),
            pl.BlockSpec((1, BLK_I, N), lambda b, i: (b, i, 0)),
        ],
        out_specs=pl.BlockSpec((1, BLK_I, OUT), lambda b, i: (b, i, 0)),
        out_shape=jax.ShapeDtypeStruct((B, N, OUT), jnp.float32),
        scratch_shapes=[pltpu.VMEM((N, OUT), jnp.float32)],
        compiler_params=pltpu.CompilerParams(
            dimension_semantics=("parallel", "arbitrary"),
        ),
    )(input, weight, bias.reshape(1, OUT), adj)
    return out


# speedup vs baseline: 1.1100x; 1.0060x over previous
import functools

import jax
import jax.numpy as jnp
from jax.experimental import pallas as pl
from jax.experimental.pallas import tpu as pltpu


def _gcn_body(x_ref, w_ref, b_ref, adj_ref, out_ref, supp_ref):
    @pl.when(pl.program_id(1) == 0)
    def _():
        supp_ref[...] = jnp.dot(
            x_ref[0].astype(jnp.bfloat16),
            w_ref[...].astype(jnp.bfloat16),
            preferred_element_type=jnp.float32,
        ).astype(jnp.bfloat16)

    out_ref[0] = (
        jnp.dot(
            adj_ref[0].astype(jnp.bfloat16),
            supp_ref[...],
            preferred_element_type=jnp.float32,
        )
        + b_ref[...]
    )


@functools.partial(jax.jit, static_argnames=())
def kernel(input, adj, weight, bias):
    B, N, IN = input.shape
    OUT = weight.shape[1]
    BLK_I = min(256, N)

    grid = (B, N // BLK_I)
    out = pl.pallas_call(
        _gcn_body,
        grid=grid,
        in_specs=[
            pl.BlockSpec((1, N, IN), lambda b, i: (b, 0, 0)),
            pl.BlockSpec((IN, OUT), lambda b, i: (0, 0)),
            pl.BlockSpec((1, OUT), lambda b, i: (0, 0)),
            pl.BlockSpec((1, BLK_I, N), lambda b, i: (b, i, 0)),
        ],
        out_specs=pl.BlockSpec((1, BLK_I, OUT), lambda b, i: (b, i, 0)),
        out_shape=jax.ShapeDtypeStruct((B, N, OUT), jnp.float32),
        scratch_shapes=[pltpu.VMEM((N, OUT), jnp.bfloat16)],
        compiler_params=pltpu.CompilerParams(
            dimension_semantics=("parallel", "arbitrary"),
        ),
    )(input, weight, bias.reshape(1, OUT), adj)
    return out
